# baseline (speedup 1.0000x reference)
import functools

import jax
import jax.numpy as jnp
from jax import lax
from jax.experimental import pallas as pl
from jax.experimental.pallas import tpu as pltpu
from jax.experimental.pallas import tpu_sc as plsc

_LANES = 16
_DROP_INF = 2**30


def _qkv_body(x_ref, w_ref, b_ref, o_ref, *, scale, nheads):
    j = pl.program_id(0)
    y = lax.dot_general(x_ref[...], w_ref[...], (((1,), (1,)), ((), ())),
                        preferred_element_type=jnp.float32)
    y = y + b_ref[0]
    y = jnp.where(j < nheads, y * scale, y)
    o_ref[...] = y[None]


def _qkv(x, wq, bq, wk, bk, wv, bv, scale, nheads, hdim, tile=256):
    t, e = x.shape
    wcat = jnp.concatenate([wq, wk, wv], axis=0)
    bcat = jnp.concatenate([bq, bk, bv]).reshape(3 * nheads, 1, hdim)
    return pl.pallas_call(
        functools.partial(_qkv_body, scale=scale, nheads=nheads),
        grid=(3 * nheads, t // tile),
        in_specs=[
            pl.BlockSpec((tile, e), lambda j, i: (i, 0)),
            pl.BlockSpec((hdim, e), lambda j, i: (j, 0)),
            pl.BlockSpec((1, 1, hdim), lambda j, i: (j, 0, 0)),
        ],
        out_specs=pl.BlockSpec((1, tile, hdim), lambda j, i: (j, i, 0)),
        out_shape=jax.ShapeDtypeStruct((3 * nheads, t, hdim), jnp.float32),
    )(x, wcat, bcat)


def _scores_body(q_ref, k_ref, sm_ref, acc_ref, *, qt, hb, pad):
    i = pl.program_id(1)
    a = lax.dot_general(q_ref[0], k_ref[0], (((1,), (1,)), ((), ())),
                        preferred_element_type=jnp.float32)
    m = jnp.max(a, axis=1, keepdims=True)
    e = jnp.exp(a - m)
    z = jnp.sum(e, axis=1, keepdims=True)
    sm = e * (1.0 / z)
    sm_ref[...] = sm[None]

    @pl.when(i == 0)
    def _():
        acc_ref[...] = jnp.zeros_like(acc_ref)

    trow = i * qt + lax.broadcasted_iota(jnp.int32, (qt, 1), 0)
    contrib = jnp.sum(jnp.where(trow < hb, sm[:, :pad], 0.0), axis=0,
                      keepdims=True)
    acc_ref[...] = acc_ref[...] + contrib[None]

    @pl.when(i == pl.num_programs(1) - 1)
    def _():
        colp = lax.broadcasted_iota(jnp.int32, (1, 1, pad), 2)
        acc_ref[...] = jnp.where(colp < hb, acc_ref[...], jnp.inf)


def _scores(qkv, nheads, hdim, hb, pad, qt=256):
    _, t, _ = qkv.shape
    s = t
    out_shapes = (
        jax.ShapeDtypeStruct((nheads, t, s), jnp.float32),
        jax.ShapeDtypeStruct((nheads, 1, pad), jnp.float32),
    )
    return pl.pallas_call(
        functools.partial(_scores_body, qt=qt, hb=hb, pad=pad),
        grid=(nheads, t // qt),
        in_specs=[
            pl.BlockSpec((1, qt, hdim), lambda h, i: (h, i, 0)),
            pl.BlockSpec((1, t, hdim), lambda h, i: (nheads + h, 0, 0)),
        ],
        out_specs=(
            pl.BlockSpec((1, qt, s), lambda h, i: (h, i, 0)),
            pl.BlockSpec((1, 1, pad), lambda h, i: (h, 0, 0)),
        ),
        out_shape=out_shapes,
    )(qkv, qkv)


def _drop_steps(sm, acc0, nheads, seq, hb, pad):
    nch = pad // _LANES
    heads_per_core = nheads // 2
    mesh = plsc.VectorSubcoreMesh(core_axis_name="c", subcore_axis_name="s")

    @functools.partial(
        pl.kernel,
        mesh=mesh,
        out_type=jax.ShapeDtypeStruct((nheads, seq), jnp.int32),
        compiler_params=pltpu.CompilerParams(needs_layout_passes=False),
        scratch_types=[
            pltpu.VMEM((4, seq), jnp.float32),
            pltpu.VMEM((pad,), jnp.int32),
            pltpu.VMEM((pad,), jnp.float32),
            pltpu.VMEM((seq,), jnp.int32),
            pltpu.SemaphoreType.DMA,
            pltpu.SemaphoreType.DMA,
            pltpu.SemaphoreType.DMA,
            pltpu.SemaphoreType.DMA,
        ],
    )
    def kfn(sm_hbm, acc0_hbm, d_hbm,
            rowbuf, col_v, sc_v, d_v, sem0, sem1, sem2, sem3):
        cid = lax.axis_index("c")
        sid = lax.axis_index("s")
        h = sid
        sems = [sem0, sem1, sem2, sem3]

        @pl.when(cid == 0)
        def _():
            pltpu.sync_copy(acc0_hbm.at[h], sc_v)
            lane = lax.iota(jnp.int32, _LANES)
            for ch in range(nch):
                col_v[pl.ds(ch * _LANES, _LANES)] = lane + ch * _LANES
            for ch in range(seq // _LANES):
                d_v[pl.ds(ch * _LANES, _LANES)] = jnp.full(
                    (_LANES,), _DROP_INF, jnp.int32)

            def start(t, b):
                pltpu.make_async_copy(sm_hbm.at[h, t], rowbuf.at[b],
                                      sems[b]).start()

            def scan_chunk(ch, scc, idc, mnv, colv, slotv):
                slv = lane + ch * _LANES
                lt = scc < mnv
                teq = (scc == mnv) & (idc > colv)
                upd = lt | teq
                colv = jnp.where(upd, idc, colv)
                slotv = jnp.where(upd, slv, slotv)
                mnv = jnp.minimum(mnv, scc)
                return mnv, colv, slotv

            def init_carry():
                mnv = jnp.full((_LANES,), jnp.inf, jnp.float32)
                colv = jnp.full((_LANES,), -1, jnp.int32)
                slotv = jnp.full((_LANES,), -1, jnp.int32)
                for ch in range(nch):
                    sl = pl.ds(ch * _LANES, _LANES)
                    mnv, colv, slotv = scan_chunk(
                        ch, sc_v[sl], col_v[sl], mnv, colv, slotv)
                return mnv, colv, slotv

            def process(t, b, carry):
                mnv, colv, slotv = carry
                mn = jnp.min(mnv)
                eqlane = mnv == jnp.full((_LANES,), mn, jnp.float32)
                drop_col = jnp.max(jnp.where(eqlane, colv, -1))
                dcv = jnp.full((_LANES,), drop_col, jnp.int32)
                drop_slot = jnp.max(
                    jnp.where(eqlane & (colv == dcv), slotv, -1))
                dsv = jnp.full((_LANES,), drop_slot, jnp.int32)
                tv = jnp.full((_LANES,), t, jnp.int32)
                lane0 = lane == 0
                plsc.store_scatter(d_v, [dcv], tv, mask=lane0)
                plsc.store_scatter(col_v, [dsv], tv, mask=lane0)
                plsc.store_scatter(sc_v, [dsv],
                                   jnp.zeros((_LANES,), jnp.float32),
                                   mask=lane0)
                pltpu.make_async_copy(sm_hbm.at[h, 0], rowbuf.at[b],
                                      sems[b]).wait()
                bv = jnp.full((_LANES,), b, jnp.int32)
                mnv = jnp.full((_LANES,), jnp.inf, jnp.float32)
                colv = jnp.full((_LANES,), -1, jnp.int32)
                slotv = jnp.full((_LANES,), -1, jnp.int32)
                for ch in range(nch):
                    sl = pl.ds(ch * _LANES, _LANES)
                    idc = col_v[sl]
                    scc = sc_v[sl] + plsc.load_gather(rowbuf, [bv, idc])
                    sc_v[sl] = scc
                    mnv, colv, slotv = scan_chunk(ch, scc, idc, mnv, colv,
                                                  slotv)
                return mnv, colv, slotv

            total = seq - hb
            nquads = total // 4
            rem = total - nquads * 4
            for b in range(4):
                start(hb + b, b)
            carry = init_carry()

            def quad(j, carry):
                u = hb + 4 * j
                for b in range(4):
                    t = u + b
                    carry = process(t, b, carry)

                    @pl.when(t + 4 < seq)
                    def _(t=t, b=b):
                        start(t + 4, b)
                return carry

            carry = lax.fori_loop(0, nquads, quad, carry)
            for r in range(rem):
                carry = process(hb + 4 * nquads + r, r, carry)
            pltpu.sync_copy(d_v, d_hbm.at[h])

    return kfn(sm, acc0)


def _attn_out_body(sm_ref, d_ref, v_ref, wo_ref, b_ref, o_ref, *, qt, hb, rb):
    i = pl.program_id(0)
    h = pl.program_id(1)
    sm = sm_ref[0]
    t = i * qt + lax.broadcasted_iota(jnp.int32, sm.shape, 0)
    s = lax.broadcasted_iota(jnp.int32, sm.shape, 1)
    d = d_ref[0]
    keep = (s <= t) & ((t < hb) | (t < d) | (s >= t - rb))
    w = jnp.where(keep, sm, 0.0)
    denom = jnp.maximum(jnp.sum(w, axis=1, keepdims=True), 1e-37)
    w = w * (1.0 / denom)
    head_out = lax.dot_general(w, v_ref[0], (((1,), (0,)), ((), ())),
                               preferred_element_type=jnp.float32)
    y = lax.dot_general(head_out, wo_ref[0], (((1,), (1,)), ((), ())),
                        preferred_element_type=jnp.float32)

    @pl.when(h == 0)
    def _():
        o_ref[...] = jnp.broadcast_to(b_ref[...], o_ref.shape)

    o_ref[...] = o_ref[...] + y


def _attn_out(sm, d, qkv, wo, bo, nheads, hdim, hb, rb, qt=256):
    _, t, s = sm.shape
    e = nheads * hdim
    wo_r = wo.reshape(e, nheads, hdim).transpose(1, 0, 2)
    return pl.pallas_call(
        functools.partial(_attn_out_body, qt=qt, hb=hb, rb=rb),
        grid=(t // qt, nheads),
        in_specs=[
            pl.BlockSpec((1, qt, s), lambda i, h: (h, i, 0)),
            pl.BlockSpec((1, 1, s), lambda i, h: (h, 0, 0)),
            pl.BlockSpec((1, t, hdim), lambda i, h: (2 * nheads + h, 0, 0)),
            pl.BlockSpec((1, e, hdim), lambda i, h: (h, 0, 0)),
            pl.BlockSpec((1, e), lambda i, h: (0, 0)),
        ],
        out_specs=pl.BlockSpec((qt, e), lambda i, h: (i, 0)),
        out_shape=jax.ShapeDtypeStruct((t, e), jnp.float32),
    )(sm, d, qkv, wo_r, bo.reshape(1, e))


def kernel(hidden_states, Wq, bq, Wk, bk, Wv, bv, Wo, bo):
    bsz, seq, embed = hidden_states.shape
    nheads = 16
    hdim = embed // nheads
    scaling = hdim ** (-0.5)
    hb = int(0.1 * seq)
    hb = hb + int(hb * 0.5)
    rb = int(0.1 * seq)
    pad = ((hb + _LANES - 1) // _LANES) * _LANES

    x = hidden_states.reshape(seq, embed)
    qkv = _qkv(x, Wq, bq, Wk, bk, Wv, bv, scaling, nheads, hdim)
    sm, acc0 = _scores(qkv, nheads, hdim, hb, pad)
    d = _drop_steps(sm, acc0.reshape(nheads, pad), nheads, seq, hb, pad)
    out = _attn_out(sm, d.reshape(nheads, 1, seq), qkv, Wo, bo,
                    nheads, hdim, hb, rb)
    return out.reshape(bsz, seq, embed)


# speedup vs baseline: 1.0135x; 1.0135x over previous
import functools

import jax
import jax.numpy as jnp
from jax import lax
from jax.experimental import pallas as pl
from jax.experimental.pallas import tpu as pltpu
from jax.experimental.pallas import tpu_sc as plsc

_LANES = 16
_DROP_INF = 2**30


def _qkv_body(x_ref, w_ref, b_ref, o_ref, *, scale, nheads):
    j = pl.program_id(0)
    y = lax.dot_general(x_ref[...], w_ref[...], (((1,), (1,)), ((), ())),
                        preferred_element_type=jnp.float32)
    y = y + b_ref[0]
    y = jnp.where(j < nheads, y * scale, y)
    o_ref[...] = y[None]


def _qkv(x, wq, bq, wk, bk, wv, bv, scale, nheads, hdim, tile=256):
    t, e = x.shape
    wcat = jnp.concatenate([wq, wk, wv], axis=0)
    bcat = jnp.concatenate([bq, bk, bv]).reshape(3 * nheads, 1, hdim)
    return pl.pallas_call(
        functools.partial(_qkv_body, scale=scale, nheads=nheads),
        grid=(3 * nheads, t // tile),
        in_specs=[
            pl.BlockSpec((tile, e), lambda j, i: (i, 0)),
            pl.BlockSpec((hdim, e), lambda j, i: (j, 0)),
            pl.BlockSpec((1, 1, hdim), lambda j, i: (j, 0, 0)),
        ],
        out_specs=pl.BlockSpec((1, tile, hdim), lambda j, i: (j, i, 0)),
        out_shape=jax.ShapeDtypeStruct((3 * nheads, t, hdim), jnp.float32),
    )(x, wcat, bcat)


def _scores_body(q_ref, k_ref, sm_ref, acc_ref, *, qt, hb, pad):
    i = pl.program_id(1)
    a = lax.dot_general(q_ref[0], k_ref[0], (((1,), (1,)), ((), ())),
                        preferred_element_type=jnp.float32)
    m = jnp.max(a, axis=1, keepdims=True)
    e = jnp.exp(a - m)
    z = jnp.sum(e, axis=1, keepdims=True)
    sm = e * (1.0 / z)
    sm_ref[...] = sm[None]

    @pl.when(i == 0)
    def _():
        acc_ref[...] = jnp.zeros_like(acc_ref)

    trow = i * qt + lax.broadcasted_iota(jnp.int32, (qt, 1), 0)
    contrib = jnp.sum(jnp.where(trow < hb, sm[:, :pad], 0.0), axis=0,
                      keepdims=True)
    acc_ref[...] = acc_ref[...] + contrib[None]

    @pl.when(i == pl.num_programs(1) - 1)
    def _():
        colp = lax.broadcasted_iota(jnp.int32, (1, 1, pad), 2)
        acc_ref[...] = jnp.where(colp < hb, acc_ref[...], jnp.inf)


def _scores(qkv, nheads, hdim, hb, pad, qt=256):
    _, t, _ = qkv.shape
    s = t
    out_shapes = (
        jax.ShapeDtypeStruct((nheads, t, s), jnp.float32),
        jax.ShapeDtypeStruct((nheads, 1, pad), jnp.float32),
    )
    return pl.pallas_call(
        functools.partial(_scores_body, qt=qt, hb=hb, pad=pad),
        grid=(nheads, t // qt),
        in_specs=[
            pl.BlockSpec((1, qt, hdim), lambda h, i: (h, i, 0)),
            pl.BlockSpec((1, t, hdim), lambda h, i: (nheads + h, 0, 0)),
        ],
        out_specs=(
            pl.BlockSpec((1, qt, s), lambda h, i: (h, i, 0)),
            pl.BlockSpec((1, 1, pad), lambda h, i: (h, 0, 0)),
        ),
        out_shape=out_shapes,
    )(qkv, qkv)


def _drop_steps(sm, acc0, nheads, seq, hb, pad):
    nch = pad // _LANES
    heads_per_core = nheads // 2
    mesh = plsc.VectorSubcoreMesh(core_axis_name="c", subcore_axis_name="s")

    @functools.partial(
        pl.kernel,
        mesh=mesh,
        out_type=jax.ShapeDtypeStruct((nheads, seq), jnp.int32),
        compiler_params=pltpu.CompilerParams(needs_layout_passes=False),
        scratch_types=[
            pltpu.VMEM((4, seq), jnp.float32),
            pltpu.VMEM((pad,), jnp.int32),
            pltpu.VMEM((pad,), jnp.float32),
            pltpu.VMEM((seq,), jnp.int32),
            pltpu.SemaphoreType.DMA,
            pltpu.SemaphoreType.DMA,
            pltpu.SemaphoreType.DMA,
            pltpu.SemaphoreType.DMA,
        ],
    )
    def kfn(sm_hbm, acc0_hbm, d_hbm,
            rowbuf, col_v, sc_v, d_v, sem0, sem1, sem2, sem3):
        cid = lax.axis_index("c")
        sid = lax.axis_index("s")
        h = sid
        sems = [sem0, sem1, sem2, sem3]

        @pl.when(cid == 0)
        def _():
            pltpu.sync_copy(acc0_hbm.at[h], sc_v)
            lane = lax.iota(jnp.int32, _LANES)
            for ch in range(nch):
                col_v[pl.ds(ch * _LANES, _LANES)] = lane + ch * _LANES
            for ch in range(seq // _LANES):
                d_v[pl.ds(ch * _LANES, _LANES)] = jnp.full(
                    (_LANES,), _DROP_INF, jnp.int32)

            def start(t, b):
                pltpu.make_async_copy(sm_hbm.at[h, t], rowbuf.at[b],
                                      sems[b]).start()

            def scan_chunk(ch, scc, idc, mnv, colv, slotv):
                slv = lane + ch * _LANES
                lt = scc < mnv
                teq = (scc == mnv) & (idc > colv)
                upd = lt | teq
                colv = jnp.where(upd, idc, colv)
                slotv = jnp.where(upd, slv, slotv)
                mnv = jnp.minimum(mnv, scc)
                return mnv, colv, slotv

            def init_carry():
                mnv = jnp.full((_LANES,), jnp.inf, jnp.float32)
                colv = jnp.full((_LANES,), -1, jnp.int32)
                slotv = jnp.full((_LANES,), -1, jnp.int32)
                for ch in range(nch):
                    sl = pl.ds(ch * _LANES, _LANES)
                    mnv, colv, slotv = scan_chunk(
                        ch, sc_v[sl], col_v[sl], mnv, colv, slotv)
                return mnv, colv, slotv

            def process(t, b, carry):
                mnv, colv, slotv = carry
                mn = jnp.min(mnv)
                eqlane = mnv == jnp.full((_LANES,), mn, jnp.float32)
                drop_col = jnp.max(jnp.where(eqlane, colv, -1))
                dcv = jnp.full((_LANES,), drop_col, jnp.int32)
                drop_slot = jnp.max(
                    jnp.where(eqlane & (colv == dcv), slotv, -1))
                dsv = jnp.full((_LANES,), drop_slot, jnp.int32)
                tv = jnp.full((_LANES,), t, jnp.int32)
                lane0 = lane == 0
                plsc.store_scatter(d_v, [dcv], tv, mask=lane0)
                plsc.store_scatter(col_v, [dsv], tv, mask=lane0)
                plsc.store_scatter(sc_v, [dsv],
                                   jnp.zeros((_LANES,), jnp.float32),
                                   mask=lane0)
                pltpu.make_async_copy(sm_hbm.at[h, 0], rowbuf.at[b],
                                      sems[b]).wait()
                bv = jnp.full((_LANES,), b, jnp.int32)
                mnv = jnp.full((_LANES,), jnp.inf, jnp.float32)
                colv = jnp.full((_LANES,), -1, jnp.int32)
                slotv = jnp.full((_LANES,), -1, jnp.int32)
                for ch in range(nch):
                    sl = pl.ds(ch * _LANES, _LANES)
                    idc = col_v[sl]
                    scc = sc_v[sl] + plsc.load_gather(rowbuf, [bv, idc])
                    sc_v[sl] = scc
                    mnv, colv, slotv = scan_chunk(ch, scc, idc, mnv, colv,
                                                  slotv)
                return mnv, colv, slotv

            total = seq - hb
            nquads = total // 4
            rem = total - nquads * 4
            for b in range(4):
                start(hb + b, b)
            carry = init_carry()

            def quad(j, carry):
                u = hb + 4 * j
                for b in range(4):
                    t = u + b
                    carry = process(t, b, carry)

                    @pl.when(t + 4 < seq)
                    def _(t=t, b=b):
                        start(t + 4, b)
                return carry

            carry = lax.fori_loop(0, nquads, quad, carry)
            for r in range(rem):
                carry = process(hb + 4 * nquads + r, r, carry)
            pltpu.sync_copy(d_v, d_hbm.at[h])

    return kfn(sm, acc0)


def _attn_out_body(sm_ref, d_ref, v_ref, wo_ref, b_ref, o_ref, *, qt, hb, rb):
    i = pl.program_id(0)
    h = pl.program_id(1)
    sm = sm_ref[0]
    t = i * qt + lax.broadcasted_iota(jnp.int32, sm.shape, 0)
    s = lax.broadcasted_iota(jnp.int32, sm.shape, 1)
    d = d_ref[0]
    keep = (s <= t) & ((t < hb) | (t < d) | (s >= t - rb))
    w = jnp.where(keep, sm, 0.0)
    denom = jnp.maximum(jnp.sum(w, axis=1, keepdims=True), 1e-37)
    w = w * (1.0 / denom)
    head_out = lax.dot_general(w.astype(jnp.bfloat16),
                               v_ref[0].astype(jnp.bfloat16),
                               (((1,), (0,)), ((), ())),
                               preferred_element_type=jnp.float32)
    y = lax.dot_general(head_out.astype(jnp.bfloat16),
                        wo_ref[0].astype(jnp.bfloat16),
                        (((1,), (1,)), ((), ())),
                        preferred_element_type=jnp.float32)

    @pl.when(h == 0)
    def _():
        o_ref[...] = jnp.broadcast_to(b_ref[...], o_ref.shape)

    o_ref[...] = o_ref[...] + y


def _attn_out(sm, d, qkv, wo, bo, nheads, hdim, hb, rb, qt=256):
    _, t, s = sm.shape
    e = nheads * hdim
    wo_r = wo.reshape(e, nheads, hdim).transpose(1, 0, 2)
    return pl.pallas_call(
        functools.partial(_attn_out_body, qt=qt, hb=hb, rb=rb),
        grid=(t // qt, nheads),
        in_specs=[
            pl.BlockSpec((1, qt, s), lambda i, h: (h, i, 0)),
            pl.BlockSpec((1, 1, s), lambda i, h: (h, 0, 0)),
            pl.BlockSpec((1, t, hdim), lambda i, h: (2 * nheads + h, 0, 0)),
            pl.BlockSpec((1, e, hdim), lambda i, h: (h, 0, 0)),
            pl.BlockSpec((1, e), lambda i, h: (0, 0)),
        ],
        out_specs=pl.BlockSpec((qt, e), lambda i, h: (i, 0)),
        out_shape=jax.ShapeDtypeStruct((t, e), jnp.float32),
    )(sm, d, qkv, wo_r, bo.reshape(1, e))


def kernel(hidden_states, Wq, bq, Wk, bk, Wv, bv, Wo, bo):
    bsz, seq, embed = hidden_states.shape
    nheads = 16
    hdim = embed // nheads
    scaling = hdim ** (-0.5)
    hb = int(0.1 * seq)
    hb = hb + int(hb * 0.5)
    rb = int(0.1 * seq)
    pad = ((hb + _LANES - 1) // _LANES) * _LANES

    x = hidden_states.reshape(seq, embed)
    qkv = _qkv(x, Wq, bq, Wk, bk, Wv, bv, scaling, nheads, hdim)
    sm, acc0 = _scores(qkv, nheads, hdim, hb, pad)
    d = _drop_steps(sm, acc0.reshape(nheads, pad), nheads, seq, hb, pad)
    out = _attn_out(sm, d.reshape(nheads, 1, seq), qkv, Wo, bo,
                    nheads, hdim, hb, rb)
    return out.reshape(bsz, seq, embed)


# speedup vs baseline: 1.1112x; 1.0965x over previous
import functools

import jax
import jax.numpy as jnp
from jax import lax
from jax.experimental import pallas as pl
from jax.experimental.pallas import tpu as pltpu
from jax.experimental.pallas import tpu_sc as plsc

_LANES = 16
_DROP_INF = 2**30


def _proj_body(x_ref, w_ref, b_ref, o_ref, *, scale):
    y = lax.dot_general(x_ref[...], w_ref[...], (((1,), (1,)), ((), ())),
                        preferred_element_type=jnp.float32)
    y = y + b_ref[0]
    if scale != 1.0:
        y = y * scale
    o_ref[...] = y[None]


def _proj_heads(x, w, b, scale, nheads, hdim, tile=256):
    t, e = x.shape
    return pl.pallas_call(
        functools.partial(_proj_body, scale=scale),
        grid=(nheads, t // tile),
        in_specs=[
            pl.BlockSpec((tile, e), lambda h, i: (i, 0)),
            pl.BlockSpec((hdim, e), lambda h, i: (h, 0)),
            pl.BlockSpec((1, 1, hdim), lambda h, i: (h, 0, 0)),
        ],
        out_specs=pl.BlockSpec((1, tile, hdim), lambda h, i: (h, i, 0)),
        out_shape=jax.ShapeDtypeStruct((nheads, t, hdim), jnp.float32),
    )(x, w, b.reshape(nheads, 1, hdim))


def _scores_body(q_ref, k_ref, sm_ref, acc_ref, *, qt, hb, pad):
    i = pl.program_id(1)
    a = lax.dot_general(q_ref[0], k_ref[0], (((1,), (1,)), ((), ())),
                        preferred_element_type=jnp.float32)
    m = jnp.max(a, axis=1, keepdims=True)
    e = jnp.exp(a - m)
    z = jnp.sum(e, axis=1, keepdims=True)
    sm = e * (1.0 / z)
    sm_ref[...] = sm[None]

    @pl.when(i == 0)
    def _():
        acc_ref[...] = jnp.zeros_like(acc_ref)

    trow = i * qt + lax.broadcasted_iota(jnp.int32, (qt, 1), 0)
    contrib = jnp.sum(jnp.where(trow < hb, sm[:, :pad], 0.0), axis=0,
                      keepdims=True)
    acc_ref[...] = acc_ref[...] + contrib[None]

    @pl.when(i == pl.num_programs(1) - 1)
    def _():
        colp = lax.broadcasted_iota(jnp.int32, (1, 1, pad), 2)
        acc_ref[...] = jnp.where(colp < hb, acc_ref[...], jnp.inf)


def _scores(q, k, nheads, hdim, hb, pad, qt=256):
    _, t, _ = q.shape
    s = t
    out_shapes = (
        jax.ShapeDtypeStruct((nheads, t, s), jnp.float32),
        jax.ShapeDtypeStruct((nheads, 1, pad), jnp.float32),
    )
    return pl.pallas_call(
        functools.partial(_scores_body, qt=qt, hb=hb, pad=pad),
        grid=(nheads, t // qt),
        in_specs=[
            pl.BlockSpec((1, qt, hdim), lambda h, i: (h, i, 0)),
            pl.BlockSpec((1, t, hdim), lambda h, i: (h, 0, 0)),
        ],
        out_specs=(
            pl.BlockSpec((1, qt, s), lambda h, i: (h, i, 0)),
            pl.BlockSpec((1, 1, pad), lambda h, i: (h, 0, 0)),
        ),
        out_shape=out_shapes,
    )(q, k)


def _drop_steps(sm, acc0, nheads, seq, hb, pad):
    nch = pad // _LANES
    heads_per_core = nheads // 2
    mesh = plsc.VectorSubcoreMesh(core_axis_name="c", subcore_axis_name="s")

    @functools.partial(
        pl.kernel,
        mesh=mesh,
        out_type=jax.ShapeDtypeStruct((nheads, seq), jnp.int32),
        compiler_params=pltpu.CompilerParams(needs_layout_passes=False),
        scratch_types=[
            pltpu.VMEM((4, seq), jnp.float32),
            pltpu.VMEM((pad,), jnp.int32),
            pltpu.VMEM((pad,), jnp.float32),
            pltpu.VMEM((seq,), jnp.int32),
            pltpu.SemaphoreType.DMA,
            pltpu.SemaphoreType.DMA,
            pltpu.SemaphoreType.DMA,
            pltpu.SemaphoreType.DMA,
        ],
    )
    def kfn(sm_hbm, acc0_hbm, d_hbm,
            rowbuf, col_v, sc_v, d_v, sem0, sem1, sem2, sem3):
        cid = lax.axis_index("c")
        sid = lax.axis_index("s")
        h = sid
        sems = [sem0, sem1, sem2, sem3]

        @pl.when(cid == 0)
        def _():
            pltpu.sync_copy(acc0_hbm.at[h], sc_v)
            lane = lax.iota(jnp.int32, _LANES)
            for ch in range(nch):
                col_v[pl.ds(ch * _LANES, _LANES)] = lane + ch * _LANES
            for ch in range(seq // _LANES):
                d_v[pl.ds(ch * _LANES, _LANES)] = jnp.full(
                    (_LANES,), _DROP_INF, jnp.int32)

            def start(t, b):
                pltpu.make_async_copy(sm_hbm.at[h, t], rowbuf.at[b],
                                      sems[b]).start()

            def scan_chunk(ch, scc, idc, mnv, colv, slotv):
                slv = lane + ch * _LANES
                lt = scc < mnv
                teq = (scc == mnv) & (idc > colv)
                upd = lt | teq
                colv = jnp.where(upd, idc, colv)
                slotv = jnp.where(upd, slv, slotv)
                mnv = jnp.minimum(mnv, scc)
                return mnv, colv, slotv

            def init_carry():
                mnv = jnp.full((_LANES,), jnp.inf, jnp.float32)
                colv = jnp.full((_LANES,), -1, jnp.int32)
                slotv = jnp.full((_LANES,), -1, jnp.int32)
                for ch in range(nch):
                    sl = pl.ds(ch * _LANES, _LANES)
                    mnv, colv, slotv = scan_chunk(
                        ch, sc_v[sl], col_v[sl], mnv, colv, slotv)
                return mnv, colv, slotv

            def process(t, b, carry):
                mnv, colv, slotv = carry
                mn = jnp.min(mnv)
                eqlane = mnv == jnp.full((_LANES,), mn, jnp.float32)
                drop_col = jnp.max(jnp.where(eqlane, colv, -1))
                dcv = jnp.full((_LANES,), drop_col, jnp.int32)
                drop_slot = jnp.max(
                    jnp.where(eqlane & (colv == dcv), slotv, -1))
                dsv = jnp.full((_LANES,), drop_slot, jnp.int32)
                tv = jnp.full((_LANES,), t, jnp.int32)
                lane0 = lane == 0
                plsc.store_scatter(d_v, [dcv], tv, mask=lane0)
                plsc.store_scatter(col_v, [dsv], tv, mask=lane0)
                plsc.store_scatter(sc_v, [dsv],
                                   jnp.zeros((_LANES,), jnp.float32),
                                   mask=lane0)
                pltpu.make_async_copy(sm_hbm.at[h, 0], rowbuf.at[b],
                                      sems[b]).wait()
                bv = jnp.full((_LANES,), b, jnp.int32)
                mnv = jnp.full((_LANES,), jnp.inf, jnp.float32)
                colv = jnp.full((_LANES,), -1, jnp.int32)
                slotv = jnp.full((_LANES,), -1, jnp.int32)
                for ch in range(nch):
                    sl = pl.ds(ch * _LANES, _LANES)
                    idc = col_v[sl]
                    scc = sc_v[sl] + plsc.load_gather(rowbuf, [bv, idc])
                    sc_v[sl] = scc
                    mnv, colv, slotv = scan_chunk(ch, scc, idc, mnv, colv,
                                                  slotv)
                return mnv, colv, slotv

            total = seq - hb
            nquads = total // 4
            rem = total - nquads * 4
            for b in range(4):
                start(hb + b, b)
            carry = init_carry()

            def quad(j, carry):
                u = hb + 4 * j
                for b in range(4):
                    t = u + b
                    carry = process(t, b, carry)

                    @pl.when(t + 4 < seq)
                    def _(t=t, b=b):
                        start(t + 4, b)
                return carry

            carry = lax.fori_loop(0, nquads, quad, carry)
            for r in range(rem):
                carry = process(hb + 4 * nquads + r, r, carry)
            pltpu.sync_copy(d_v, d_hbm.at[h])

    return kfn(sm, acc0)


def _attn_out_body(sm_ref, d_ref, v_ref, wo_ref, b_ref, o_ref, *, qt, hb, rb):
    i = pl.program_id(0)
    h = pl.program_id(1)
    sm = sm_ref[0]
    t = i * qt + lax.broadcasted_iota(jnp.int32, sm.shape, 0)
    s = lax.broadcasted_iota(jnp.int32, sm.shape, 1)
    d = d_ref[0]
    keep = (s <= t) & ((t < hb) | (t < d) | (s >= t - rb))
    w = jnp.where(keep, sm, 0.0)
    denom = jnp.maximum(jnp.sum(w, axis=1, keepdims=True), 1e-37)
    w = w * (1.0 / denom)
    head_out = lax.dot_general(w.astype(jnp.bfloat16),
                               v_ref[0].astype(jnp.bfloat16),
                               (((1,), (0,)), ((), ())),
                               preferred_element_type=jnp.float32)
    y = lax.dot_general(head_out.astype(jnp.bfloat16),
                        wo_ref[0].astype(jnp.bfloat16),
                        (((1,), (1,)), ((), ())),
                        preferred_element_type=jnp.float32)

    @pl.when(h == 0)
    def _():
        o_ref[...] = jnp.broadcast_to(b_ref[...], o_ref.shape)

    o_ref[...] = o_ref[...] + y


def _attn_out(sm, d, v, wo, bo, nheads, hdim, hb, rb, qt=256):
    _, t, s = sm.shape
    e = nheads * hdim
    wo_r = wo.reshape(e, nheads, hdim).transpose(1, 0, 2)
    return pl.pallas_call(
        functools.partial(_attn_out_body, qt=qt, hb=hb, rb=rb),
        grid=(t // qt, nheads),
        in_specs=[
            pl.BlockSpec((1, qt, s), lambda i, h: (h, i, 0)),
            pl.BlockSpec((1, 1, s), lambda i, h: (h, 0, 0)),
            pl.BlockSpec((1, t, hdim), lambda i, h: (h, 0, 0)),
            pl.BlockSpec((1, e, hdim), lambda i, h: (h, 0, 0)),
            pl.BlockSpec((1, e), lambda i, h: (0, 0)),
        ],
        out_specs=pl.BlockSpec((qt, e), lambda i, h: (i, 0)),
        out_shape=jax.ShapeDtypeStruct((t, e), jnp.float32),
    )(sm, d, v, wo_r, bo.reshape(1, e))


def kernel(hidden_states, Wq, bq, Wk, bk, Wv, bv, Wo, bo):
    bsz, seq, embed = hidden_states.shape
    nheads = 16
    hdim = embed // nheads
    scaling = hdim ** (-0.5)
    hb = int(0.1 * seq)
    hb = hb + int(hb * 0.5)
    rb = int(0.1 * seq)
    pad = ((hb + _LANES - 1) // _LANES) * _LANES

    x = hidden_states.reshape(seq, embed)
    q = _proj_heads(x, Wq, bq, scaling, nheads, hdim)
    k = _proj_heads(x, Wk, bk, 1.0, nheads, hdim)
    v = _proj_heads(x, Wv, bv, 1.0, nheads, hdim)
    sm, acc0 = _scores(q, k, nheads, hdim, hb, pad)
    d = _drop_steps(sm, acc0.reshape(nheads, pad), nheads, seq, hb, pad)
    out = _attn_out(sm, d.reshape(nheads, 1, seq), v, Wo, bo,
                    nheads, hdim, hb, rb)
    return out.reshape(bsz, seq, embed)


# speedup vs baseline: 1.1842x; 1.0656x over previous
import functools

import jax
import jax.numpy as jnp
from jax import lax
from jax.experimental import pallas as pl
from jax.experimental.pallas import tpu as pltpu
from jax.experimental.pallas import tpu_sc as plsc

_LANES = 16
_DROP_INF = 2**30


def _proj_body(x_ref, w_ref, b_ref, o_ref, *, scale):
    y = lax.dot_general(x_ref[...], w_ref[...], (((1,), (1,)), ((), ())),
                        preferred_element_type=jnp.float32)
    y = y + b_ref[0]
    if scale != 1.0:
        y = y * scale
    o_ref[...] = y[None]


def _proj_heads(x, w, b, scale, nheads, hdim, tile=256):
    t, e = x.shape
    return pl.pallas_call(
        functools.partial(_proj_body, scale=scale),
        grid=(nheads, t // tile),
        in_specs=[
            pl.BlockSpec((tile, e), lambda h, i: (i, 0)),
            pl.BlockSpec((hdim, e), lambda h, i: (h, 0)),
            pl.BlockSpec((1, 1, hdim), lambda h, i: (h, 0, 0)),
        ],
        out_specs=pl.BlockSpec((1, tile, hdim), lambda h, i: (h, i, 0)),
        out_shape=jax.ShapeDtypeStruct((nheads, t, hdim), jnp.float32),
    )(x, w, b.reshape(nheads, 1, hdim))


def _scores_body(q_ref, k_ref, sm_ref, acc_ref, *, qt, hb, pad):
    i = pl.program_id(1)
    a = lax.dot_general(q_ref[0], k_ref[0], (((1,), (1,)), ((), ())),
                        preferred_element_type=jnp.float32)
    m = jnp.max(a, axis=1, keepdims=True)
    e = jnp.exp(a - m)
    z = jnp.sum(e, axis=1, keepdims=True)
    sm = e * (1.0 / z)
    sm_ref[...] = sm[None]

    @pl.when(i == 0)
    def _():
        acc_ref[...] = jnp.zeros_like(acc_ref)

    trow = i * qt + lax.broadcasted_iota(jnp.int32, (qt, 1), 0)
    contrib = jnp.sum(jnp.where(trow < hb, sm[:, :pad], 0.0), axis=0,
                      keepdims=True)
    acc_ref[...] = acc_ref[...] + contrib[None]

    @pl.when(i == pl.num_programs(1) - 1)
    def _():
        colp = lax.broadcasted_iota(jnp.int32, (1, 1, pad), 2)
        acc_ref[...] = jnp.where(colp < hb, acc_ref[...], jnp.inf)


def _scores(q, k, nheads, hdim, hb, pad, qt=512):
    _, t, _ = q.shape
    s = t
    out_shapes = (
        jax.ShapeDtypeStruct((nheads, t, s), jnp.float32),
        jax.ShapeDtypeStruct((nheads, 1, pad), jnp.float32),
    )
    return pl.pallas_call(
        functools.partial(_scores_body, qt=qt, hb=hb, pad=pad),
        grid=(nheads, t // qt),
        in_specs=[
            pl.BlockSpec((1, qt, hdim), lambda h, i: (h, i, 0)),
            pl.BlockSpec((1, t, hdim), lambda h, i: (h, 0, 0)),
        ],
        out_specs=(
            pl.BlockSpec((1, qt, s), lambda h, i: (h, i, 0)),
            pl.BlockSpec((1, 1, pad), lambda h, i: (h, 0, 0)),
        ),
        out_shape=out_shapes,
    )(q, k)


def _drop_steps(sm, acc0, nheads, seq, hb, pad):
    nch = pad // _LANES
    heads_per_core = nheads // 2
    mesh = plsc.VectorSubcoreMesh(core_axis_name="c", subcore_axis_name="s")

    @functools.partial(
        pl.kernel,
        mesh=mesh,
        out_type=jax.ShapeDtypeStruct((nheads, seq), jnp.int32),
        compiler_params=pltpu.CompilerParams(needs_layout_passes=False),
        scratch_types=[
            pltpu.VMEM((4, seq), jnp.float32),
            pltpu.VMEM((pad,), jnp.int32),
            pltpu.VMEM((pad,), jnp.float32),
            pltpu.VMEM((seq,), jnp.int32),
            pltpu.SemaphoreType.DMA,
            pltpu.SemaphoreType.DMA,
            pltpu.SemaphoreType.DMA,
            pltpu.SemaphoreType.DMA,
        ],
    )
    def kfn(sm_hbm, acc0_hbm, d_hbm,
            rowbuf, col_v, sc_v, d_v, sem0, sem1, sem2, sem3):
        cid = lax.axis_index("c")
        sid = lax.axis_index("s")
        h = sid
        sems = [sem0, sem1, sem2, sem3]

        @pl.when(cid == 0)
        def _():
            pltpu.sync_copy(acc0_hbm.at[h], sc_v)
            lane = lax.iota(jnp.int32, _LANES)
            for ch in range(nch):
                col_v[pl.ds(ch * _LANES, _LANES)] = lane + ch * _LANES
            for ch in range(seq // _LANES):
                d_v[pl.ds(ch * _LANES, _LANES)] = jnp.full(
                    (_LANES,), _DROP_INF, jnp.int32)

            def start(t, b):
                pltpu.make_async_copy(sm_hbm.at[h, t], rowbuf.at[b],
                                      sems[b]).start()

            def scan_chunk(ch, scc, idc, mnv, colv, slotv):
                slv = lane + ch * _LANES
                lt = scc < mnv
                teq = (scc == mnv) & (idc > colv)
                upd = lt | teq
                colv = jnp.where(upd, idc, colv)
                slotv = jnp.where(upd, slv, slotv)
                mnv = jnp.minimum(mnv, scc)
                return mnv, colv, slotv

            def init_carry():
                mnv = jnp.full((_LANES,), jnp.inf, jnp.float32)
                colv = jnp.full((_LANES,), -1, jnp.int32)
                slotv = jnp.full((_LANES,), -1, jnp.int32)
                for ch in range(nch):
                    sl = pl.ds(ch * _LANES, _LANES)
                    mnv, colv, slotv = scan_chunk(
                        ch, sc_v[sl], col_v[sl], mnv, colv, slotv)
                return mnv, colv, slotv

            def process(t, b, carry):
                mnv, colv, slotv = carry
                mn = jnp.min(mnv)
                eqlane = mnv == jnp.full((_LANES,), mn, jnp.float32)
                drop_col = jnp.max(jnp.where(eqlane, colv, -1))
                dcv = jnp.full((_LANES,), drop_col, jnp.int32)
                drop_slot = jnp.max(
                    jnp.where(eqlane & (colv == dcv), slotv, -1))
                dsv = jnp.full((_LANES,), drop_slot, jnp.int32)
                tv = jnp.full((_LANES,), t, jnp.int32)
                lane0 = lane == 0
                plsc.store_scatter(d_v, [dcv], tv, mask=lane0)
                plsc.store_scatter(col_v, [dsv], tv, mask=lane0)
                plsc.store_scatter(sc_v, [dsv],
                                   jnp.zeros((_LANES,), jnp.float32),
                                   mask=lane0)
                pltpu.make_async_copy(sm_hbm.at[h, 0], rowbuf.at[b],
                                      sems[b]).wait()
                bv = jnp.full((_LANES,), b, jnp.int32)
                mnv = jnp.full((_LANES,), jnp.inf, jnp.float32)
                colv = jnp.full((_LANES,), -1, jnp.int32)
                slotv = jnp.full((_LANES,), -1, jnp.int32)
                for ch in range(nch):
                    sl = pl.ds(ch * _LANES, _LANES)
                    idc = col_v[sl]
                    scc = sc_v[sl] + plsc.load_gather(rowbuf, [bv, idc])
                    sc_v[sl] = scc
                    mnv, colv, slotv = scan_chunk(ch, scc, idc, mnv, colv,
                                                  slotv)
                return mnv, colv, slotv

            total = seq - hb
            nquads = total // 4
            rem = total - nquads * 4
            for b in range(4):
                start(hb + b, b)
            carry = init_carry()

            def quad(j, carry):
                u = hb + 4 * j
                for b in range(4):
                    t = u + b
                    carry = process(t, b, carry)

                    @pl.when(t + 4 < seq)
                    def _(t=t, b=b):
                        start(t + 4, b)
                return carry

            carry = lax.fori_loop(0, nquads, quad, carry)
            for r in range(rem):
                carry = process(hb + 4 * nquads + r, r, carry)
            pltpu.sync_copy(d_v, d_hbm.at[h])

    return kfn(sm, acc0)


def _attn_out_body(sm_ref, d_ref, v_ref, wo_ref, b_ref, o_ref, *, qt, hb, rb):
    i = pl.program_id(0)
    h = pl.program_id(1)
    sm = sm_ref[0]
    t = i * qt + lax.broadcasted_iota(jnp.int32, sm.shape, 0)
    s = lax.broadcasted_iota(jnp.int32, sm.shape, 1)
    d = d_ref[0]
    keep = (s <= t) & ((t < hb) | (t < d) | (s >= t - rb))
    w = jnp.where(keep, sm, 0.0)
    denom = jnp.maximum(jnp.sum(w, axis=1, keepdims=True), 1e-37)
    w = w * (1.0 / denom)
    head_out = lax.dot_general(w.astype(jnp.bfloat16),
                               v_ref[0].astype(jnp.bfloat16),
                               (((1,), (0,)), ((), ())),
                               preferred_element_type=jnp.float32)
    y = lax.dot_general(head_out.astype(jnp.bfloat16),
                        wo_ref[0].astype(jnp.bfloat16),
                        (((1,), (1,)), ((), ())),
                        preferred_element_type=jnp.float32)

    @pl.when(h == 0)
    def _():
        o_ref[...] = jnp.broadcast_to(b_ref[...], o_ref.shape)

    o_ref[...] = o_ref[...] + y


def _attn_out(sm, d, v, wo, bo, nheads, hdim, hb, rb, qt=512):
    _, t, s = sm.shape
    e = nheads * hdim
    wo_r = wo.reshape(e, nheads, hdim).transpose(1, 0, 2)
    return pl.pallas_call(
        functools.partial(_attn_out_body, qt=qt, hb=hb, rb=rb),
        grid=(t // qt, nheads),
        in_specs=[
            pl.BlockSpec((1, qt, s), lambda i, h: (h, i, 0)),
            pl.BlockSpec((1, 1, s), lambda i, h: (h, 0, 0)),
            pl.BlockSpec((1, t, hdim), lambda i, h: (h, 0, 0)),
            pl.BlockSpec((1, e, hdim), lambda i, h: (h, 0, 0)),
            pl.BlockSpec((1, e), lambda i, h: (0, 0)),
        ],
        out_specs=pl.BlockSpec((qt, e), lambda i, h: (i, 0)),
        out_shape=jax.ShapeDtypeStruct((t, e), jnp.float32),
    )(sm, d, v, wo_r, bo.reshape(1, e))


def kernel(hidden_states, Wq, bq, Wk, bk, Wv, bv, Wo, bo):
    bsz, seq, embed = hidden_states.shape
    nheads = 16
    hdim = embed // nheads
    scaling = hdim ** (-0.5)
    hb = int(0.1 * seq)
    hb = hb + int(hb * 0.5)
    rb = int(0.1 * seq)
    pad = ((hb + _LANES - 1) // _LANES) * _LANES

    x = hidden_states.reshape(seq, embed)
    q = _proj_heads(x, Wq, bq, scaling, nheads, hdim)
    k = _proj_heads(x, Wk, bk, 1.0, nheads, hdim)
    v = _proj_heads(x, Wv, bv, 1.0, nheads, hdim)
    sm, acc0 = _scores(q, k, nheads, hdim, hb, pad)
    d = _drop_steps(sm, acc0.reshape(nheads, pad), nheads, seq, hb, pad)
    out = _attn_out(sm, d.reshape(nheads, 1, seq), v, Wo, bo,
                    nheads, hdim, hb, rb)
    return out.reshape(bsz, seq, embed)
